# baseline (speedup 1.0000x reference)
import functools

import jax
import jax.numpy as jnp
from jax import lax
from jax.experimental import pallas as pl
from jax.experimental.pallas import tpu as pltpu
from jax.experimental.pallas import tpu_sc as plsc

N = 10000
E = 320000
H = 128
F = 128
G = 256

NC = 2
NS = 16
NW = NC * NS
CHUNK = 128
NCH = 2560
EPAD = NCH * CHUNK
FAST_CID = 0
CF = 128
CS = 32
NCH_PAD = NCH
NPAD = N
NSRC = N + 8
RPT = 624
RTAIL = NPAD - NS * RPT
SHIFT = 14
MASK = (1 << SHIFT) - 1

_f32 = jnp.float32


def _edge_aggr_body(x_hbm, epk_hbm, zeros_hbm, out_hbm,
                    pbuf, sidx0, didx0, sidx1, didx1, rows0, rows1,
                    aggr, sem):
    cid = lax.axis_index("c")
    sid = lax.axis_index("s")
    wid = cid * NS + sid
    r0 = pl.multiple_of(sid * RPT, 8)
    pltpu.sync_copy(zeros_hbm.at[pl.ds(r0, RPT)], aggr.at[pl.ds(r0, RPT)])

    @pl.when(sid == NS - 1)
    def _():
        pltpu.sync_copy(zeros_hbm.at[pl.ds(NS * RPT, RTAIL)],
                        aggr.at[pl.ds(NS * RPT, RTAIL)])

    nch = jnp.where(cid == FAST_CID, CF, CS)
    cbase = pl.multiple_of(
        jnp.where(cid == FAST_CID, NS * CS + sid * CF, sid * CS), 8)
    pltpu.sync_copy(epk_hbm.at[pl.ds(cbase, CF)], pbuf)

    def unpack(c, sidx, didx):
        def step(t, carry):
            k = t * 16
            v = pbuf[c, pl.ds(k, 16)]
            sidx[pl.ds(k, 16)] = lax.bitwise_and(v, MASK)
            didx[pl.ds(k, 16)] = lax.shift_right_logical(v, SHIFT)
            return carry

        lax.fori_loop(0, CHUNK // 16, step, 0)

    plsc.subcore_barrier()

    unpack(0, sidx0, didx0)
    pltpu.async_copy(x_hbm.at[sidx0], rows0, sem)

    def body(i, carry):
        j0 = i * 2
        j1 = j0 + 1
        unpack(j1, sidx1, didx1)
        pltpu.make_async_copy(x_hbm.at[sidx0], rows0, sem).wait()
        pltpu.async_copy(x_hbm.at[sidx1], rows1, sem)
        pltpu.sync_copy(rows0, aggr.at[didx0], add=True)
        unpack(jnp.minimum(j1 + 1, nch - 1), sidx0, didx0)
        pltpu.make_async_copy(x_hbm.at[sidx1], rows1, sem).wait()
        pltpu.async_copy(x_hbm.at[sidx0], rows0, sem)
        pltpu.sync_copy(rows1, aggr.at[didx1], add=True)
        return carry

    lax.fori_loop(0, nch // 2, body, 0)
    pltpu.make_async_copy(x_hbm.at[sidx0], rows0, sem).wait()

    plsc.subcore_barrier()
    pltpu.sync_copy(aggr.at[pl.ds(r0, RPT)], out_hbm.at[cid, pl.ds(r0, RPT)])

    @pl.when(sid == NS - 1)
    def _():
        pltpu.sync_copy(aggr.at[pl.ds(NS * RPT, RTAIL)],
                        out_hbm.at[cid, pl.ds(NS * RPT, RTAIL)])


@functools.cache
def _edge_aggr_kernel():
    return pl.kernel(
        _edge_aggr_body,
        out_type=jax.ShapeDtypeStruct((NC, NPAD, H), _f32),
        mesh=plsc.VectorSubcoreMesh(core_axis_name="c", subcore_axis_name="s",
                                    num_cores=NC, num_subcores=NS),
        scratch_types=[
            pltpu.VMEM((CF, CHUNK), jnp.int32),
            pltpu.VMEM((CHUNK,), jnp.int32),
            pltpu.VMEM((CHUNK,), jnp.int32),
            pltpu.VMEM((CHUNK,), jnp.int32),
            pltpu.VMEM((CHUNK,), jnp.int32),
            pltpu.VMEM((CHUNK, H), _f32),
            pltpu.VMEM((CHUNK, H), _f32),
            pltpu.VMEM_SHARED((NPAD, H), _f32),
            pltpu.SemaphoreType.DMA,
        ],
    )


def _edge_aggr(x, epk, zeros):
    return _edge_aggr_kernel()(x, epk, zeros)


_RB = 1000
_NB = N // _RB


def _x0_body(z_ref, pos_ref, emb_ref, wpos_ref, bpos_ref, wt_ref, wb_ref,
             bc_ref, o_ref):
    zcol = z_ref[...]
    cols = lax.broadcasted_iota(jnp.int32, (_RB, 128), 1)
    onehot = (cols == zcol).astype(_f32)
    atom = jnp.dot(onehot, emb_ref[...],
                   preferred_element_type=_f32,
                   precision=lax.Precision.HIGHEST)
    pe = jnp.dot(pos_ref[...], wpos_ref[...],
                 preferred_element_type=_f32) + bpos_ref[...]
    acc = jnp.dot(atom, wt_ref[...], preferred_element_type=_f32)
    acc += jnp.dot(pe, wb_ref[...], preferred_element_type=_f32)
    o_ref[...] = jnp.maximum(acc + bc_ref[...], 0.0)


def _x0_call(z2, pos8, emb_pad, wpos8, bpos2, wt, wb, bc2):
    return pl.pallas_call(
        _x0_body,
        grid=(_NB,),
        in_specs=[
            pl.BlockSpec((_RB, 1), lambda i: (i, 0)),
            pl.BlockSpec((_RB, 8), lambda i: (i, 0)),
            pl.BlockSpec((128, H), lambda i: (0, 0)),
            pl.BlockSpec((8, H), lambda i: (0, 0)),
            pl.BlockSpec((1, H), lambda i: (0, 0)),
            pl.BlockSpec((H, H), lambda i: (0, 0)),
            pl.BlockSpec((H, H), lambda i: (0, 0)),
            pl.BlockSpec((1, H), lambda i: (0, 0)),
        ],
        out_specs=pl.BlockSpec((_RB, H), lambda i: (i, 0)),
        out_shape=jax.ShapeDtypeStruct((N, H), _f32),
    )(z2, pos8, emb_pad, wpos8, bpos2, wt, wb, bc2)


def _mlp_body(x_ref, p_ref, w1_ref, b1_ref, w2_ref, b2_ref, o_ref, *,
              final_relu):
    h = x_ref[...] + p_ref[0] + p_ref[1]
    h = jnp.maximum(
        jnp.dot(h, w1_ref[...], preferred_element_type=_f32) + b1_ref[...],
        0.0)
    o = jnp.dot(h, w2_ref[...], preferred_element_type=_f32) + b2_ref[...]
    if final_relu:
        o = jnp.maximum(o, 0.0)
    o_ref[...] = o


def _mlp_call(x, parts, w1, b12, w2, b22, final_relu):
    return pl.pallas_call(
        functools.partial(_mlp_body, final_relu=final_relu),
        grid=(_NB,),
        in_specs=[
            pl.BlockSpec((_RB, H), lambda i: (i, 0)),
            pl.BlockSpec((NC, _RB, H), lambda i: (0, i, 0)),
            pl.BlockSpec((H, H), lambda i: (0, 0)),
            pl.BlockSpec((1, H), lambda i: (0, 0)),
            pl.BlockSpec((H, H), lambda i: (0, 0)),
            pl.BlockSpec((1, H), lambda i: (0, 0)),
        ],
        out_specs=pl.BlockSpec((_RB, H), lambda i: (i, 0)),
        out_shape=jax.ShapeDtypeStruct((N, H), _f32),
    )(x, parts, w1, b12, w2, b22)


def _pool_body(x_ref, b3_ref, wp1_ref, bp1_ref, wp2_ref, bp2_ref, o_ref,
               acc_ref):
    i = pl.program_id(0)

    @pl.when(i == 0)
    def _():
        acc_ref[...] = jnp.zeros_like(acc_ref)

    brow = b3_ref[0]
    rows = lax.broadcasted_iota(jnp.int32, (G, _RB), 0)
    oh = (rows == brow).astype(_f32)
    acc_ref[...] += jnp.dot(oh, x_ref[...], preferred_element_type=_f32,
                            precision=lax.Precision.HIGHEST)

    @pl.when(i == _NB - 1)
    def _():
        g = acc_ref[...]
        h = jnp.maximum(
            jnp.dot(g, wp1_ref[...], preferred_element_type=_f32)
            + bp1_ref[...], 0.0)
        o_ref[...] = jnp.dot(h, wp2_ref[...], preferred_element_type=_f32
                             ) + bp2_ref[...]


def _pool_call(x, batch3, wp1, bp12, wp2pad, bp2pad):
    return pl.pallas_call(
        _pool_body,
        grid=(_NB,),
        in_specs=[
            pl.BlockSpec((_RB, H), lambda i: (i, 0)),
            pl.BlockSpec((1, 1, _RB), lambda i: (i, 0, 0)),
            pl.BlockSpec((F, F), lambda i: (0, 0)),
            pl.BlockSpec((1, F), lambda i: (0, 0)),
            pl.BlockSpec((F, F), lambda i: (0, 0)),
            pl.BlockSpec((1, F), lambda i: (0, 0)),
        ],
        out_specs=pl.BlockSpec((G, F), lambda i: (0, 0)),
        out_shape=jax.ShapeDtypeStruct((G, F), _f32),
        scratch_shapes=[pltpu.VMEM((G, F), _f32)],
    )(x, batch3, wp1, bp12, wp2pad, bp2pad)


def kernel(z, pos, edge_index, batch, embed, W_pos, b_pos, W_comb, b_comb,
           gin0_W1, gin0_b1, gin0_W2, gin0_b2,
           gin1_W1, gin1_b1, gin1_W2, gin1_b2,
           gin2_W1, gin2_b1, gin2_W2, gin2_b2,
           Wp1, bp1, Wp2, bp2):
    src = edge_index[0].astype(jnp.int32)
    dst = edge_index[1].astype(jnp.int32)
    pad = EPAD - E
    packed = jnp.left_shift(dst, SHIFT) | src
    pad_dst = jnp.arange(pad, dtype=jnp.int32) % N
    pad_packed = jnp.left_shift(pad_dst, SHIFT) | N
    epk = jnp.concatenate([packed, pad_packed]).reshape(NCH, CHUNK)
    epk = jnp.pad(epk, ((0, NCH_PAD - NCH), (0, 0)))
    z2 = z.astype(jnp.int32).reshape(N, 1)
    pos8 = jnp.pad(pos, ((0, 0), (0, 8 - pos.shape[1])))
    emb_pad = jnp.pad(embed, ((0, 128 - embed.shape[0]), (0, 0)))
    wpos8 = jnp.pad(W_pos, ((0, 8 - W_pos.shape[0]), (0, 0)))
    zeros = jnp.zeros((NPAD, H), _f32)
    batch3 = batch.astype(jnp.int32).reshape(_NB, 1, _RB)
    wp2pad = jnp.pad(Wp2, ((0, 0), (0, F - Wp2.shape[1])))
    bp2pad = jnp.pad(bp2, ((0, F - bp2.shape[0]),)).reshape(1, F)

    x = _x0_call(z2, pos8, emb_pad, wpos8, b_pos.reshape(1, H),
                 W_comb[:H], W_comb[H:], b_comb.reshape(1, H))

    gin = [(gin0_W1, gin0_b1, gin0_W2, gin0_b2),
           (gin1_W1, gin1_b1, gin1_W2, gin1_b2),
           (gin2_W1, gin2_b1, gin2_W2, gin2_b2)]
    zrows = jnp.zeros((NSRC - N, H), _f32)
    for l, (w1, b1, w2, b2) in enumerate(gin):
        parts = _edge_aggr(jnp.concatenate([x, zrows]), epk, zeros)
        x = _mlp_call(x, parts, w1, b1.reshape(1, H), w2, b2.reshape(1, H),
                      final_relu=(l < 2))

    out = _pool_call(x, batch3, Wp1, bp1.reshape(1, F), wp2pad, bp2pad)
    return out[:, :1]


# speedup vs baseline: 3.3623x; 3.3623x over previous
import functools

import jax
import jax.numpy as jnp
from jax import lax
from jax.experimental import pallas as pl
from jax.experimental.pallas import tpu as pltpu
from jax.experimental.pallas import tpu_sc as plsc

N = 10000
E = 320000
H = 128
F = 128
G = 256

NC = 2
NS = 16
NW = NC * NS
CHUNK = 128
NCH = 2560
EPAD = NCH * CHUNK
CPT = NCH // NW
NSINK = 128
NPAD = N + NSINK
ZPT = 632
ZTAIL = NPAD - NS * ZPT
WPT = 624
WTAIL = N - NS * WPT
SHIFT = 14
MASK = (1 << SHIFT) - 1

_f32 = jnp.float32


def _edge_aggr_body(x_hbm, epk_hbm, zeros_hbm, out_hbm,
                    pbuf, sidx0, didx0, sidx1, didx1, rows0, rows1,
                    aggr, sem):
    cid = lax.axis_index("c")
    sid = lax.axis_index("s")
    wid = cid * NS + sid
    z0 = pl.multiple_of(sid * ZPT, 8)
    pltpu.sync_copy(zeros_hbm.at[pl.ds(z0, ZPT)], aggr.at[pl.ds(z0, ZPT)])

    @pl.when(sid == NS - 1)
    def _():
        pltpu.sync_copy(zeros_hbm.at[pl.ds(NS * ZPT, ZTAIL)],
                        aggr.at[pl.ds(NS * ZPT, ZTAIL)])

    nch = CPT
    cbase = pl.multiple_of(wid * CPT, 8)
    pltpu.sync_copy(epk_hbm.at[pl.ds(cbase, CPT)], pbuf)

    def unpack(c, sidx, didx):
        def step(t, carry):
            k = t * 16
            v = pbuf[c, pl.ds(k, 16)]
            sidx[pl.ds(k, 16)] = lax.bitwise_and(v, MASK)
            didx[pl.ds(k, 16)] = lax.shift_right_logical(v, SHIFT)
            return carry

        lax.fori_loop(0, CHUNK // 16, step, 0)

    plsc.subcore_barrier()

    unpack(0, sidx0, didx0)
    pltpu.async_copy(x_hbm.at[sidx0], rows0, sem)

    def body(i, carry):
        j0 = i * 2
        j1 = j0 + 1
        unpack(j1, sidx1, didx1)
        pltpu.make_async_copy(x_hbm.at[sidx0], rows0, sem).wait()
        pltpu.async_copy(x_hbm.at[sidx1], rows1, sem)
        pltpu.sync_copy(rows0, aggr.at[didx0], add=True)
        unpack(jnp.minimum(j1 + 1, nch - 1), sidx0, didx0)
        pltpu.make_async_copy(x_hbm.at[sidx1], rows1, sem).wait()
        pltpu.async_copy(x_hbm.at[sidx0], rows0, sem)
        pltpu.sync_copy(rows1, aggr.at[didx1], add=True)
        return carry

    lax.fori_loop(0, nch // 2, body, 0)
    pltpu.make_async_copy(x_hbm.at[sidx0], rows0, sem).wait()

    plsc.subcore_barrier()
    w0 = pl.multiple_of(sid * WPT, 8)
    pltpu.sync_copy(aggr.at[pl.ds(w0, WPT)], out_hbm.at[cid, pl.ds(w0, WPT)])

    @pl.when(sid == NS - 1)
    def _():
        pltpu.sync_copy(aggr.at[pl.ds(NS * WPT, WTAIL)],
                        out_hbm.at[cid, pl.ds(NS * WPT, WTAIL)])


@functools.cache
def _edge_aggr_kernel():
    return pl.kernel(
        _edge_aggr_body,
        out_type=jax.ShapeDtypeStruct((NC, N, H), _f32),
        mesh=plsc.VectorSubcoreMesh(core_axis_name="c", subcore_axis_name="s",
                                    num_cores=NC, num_subcores=NS),
        scratch_types=[
            pltpu.VMEM((CPT, CHUNK), jnp.int32),
            pltpu.VMEM((CHUNK,), jnp.int32),
            pltpu.VMEM((CHUNK,), jnp.int32),
            pltpu.VMEM((CHUNK,), jnp.int32),
            pltpu.VMEM((CHUNK,), jnp.int32),
            pltpu.VMEM((CHUNK, H), _f32),
            pltpu.VMEM((CHUNK, H), _f32),
            pltpu.VMEM_SHARED((NPAD, H), _f32),
            pltpu.SemaphoreType.DMA,
        ],
    )


def _edge_aggr(x, epk, zeros):
    return _edge_aggr_kernel()(x, epk, zeros)


_RB = 1000
_NB = N // _RB


def _x0_body(z_ref, pos_ref, emb_ref, wpos_ref, bpos_ref, wt_ref, wb_ref,
             bc_ref, o_ref):
    zcol = z_ref[...]
    cols = lax.broadcasted_iota(jnp.int32, (_RB, 128), 1)
    onehot = (cols == zcol).astype(_f32)
    atom = jnp.dot(onehot, emb_ref[...],
                   preferred_element_type=_f32,
                   precision=lax.Precision.HIGHEST)
    pe = jnp.dot(pos_ref[...], wpos_ref[...],
                 preferred_element_type=_f32) + bpos_ref[...]
    acc = jnp.dot(atom, wt_ref[...], preferred_element_type=_f32)
    acc += jnp.dot(pe, wb_ref[...], preferred_element_type=_f32)
    o_ref[...] = jnp.maximum(acc + bc_ref[...], 0.0)


def _x0_call(z2, pos8, emb_pad, wpos8, bpos2, wt, wb, bc2):
    return pl.pallas_call(
        _x0_body,
        grid=(_NB,),
        in_specs=[
            pl.BlockSpec((_RB, 1), lambda i: (i, 0)),
            pl.BlockSpec((_RB, 8), lambda i: (i, 0)),
            pl.BlockSpec((128, H), lambda i: (0, 0)),
            pl.BlockSpec((8, H), lambda i: (0, 0)),
            pl.BlockSpec((1, H), lambda i: (0, 0)),
            pl.BlockSpec((H, H), lambda i: (0, 0)),
            pl.BlockSpec((H, H), lambda i: (0, 0)),
            pl.BlockSpec((1, H), lambda i: (0, 0)),
        ],
        out_specs=pl.BlockSpec((_RB, H), lambda i: (i, 0)),
        out_shape=jax.ShapeDtypeStruct((N, H), _f32),
    )(z2, pos8, emb_pad, wpos8, bpos2, wt, wb, bc2)


def _mlp_body(x_ref, p_ref, w1_ref, b1_ref, w2_ref, b2_ref, o_ref, *,
              final_relu):
    h = x_ref[...] + p_ref[0] + p_ref[1]
    h = jnp.maximum(
        jnp.dot(h, w1_ref[...], preferred_element_type=_f32) + b1_ref[...],
        0.0)
    o = jnp.dot(h, w2_ref[...], preferred_element_type=_f32) + b2_ref[...]
    if final_relu:
        o = jnp.maximum(o, 0.0)
    o_ref[...] = o


def _mlp_call(x, parts, w1, b12, w2, b22, final_relu):
    return pl.pallas_call(
        functools.partial(_mlp_body, final_relu=final_relu),
        grid=(_NB,),
        in_specs=[
            pl.BlockSpec((_RB, H), lambda i: (i, 0)),
            pl.BlockSpec((NC, _RB, H), lambda i: (0, i, 0)),
            pl.BlockSpec((H, H), lambda i: (0, 0)),
            pl.BlockSpec((1, H), lambda i: (0, 0)),
            pl.BlockSpec((H, H), lambda i: (0, 0)),
            pl.BlockSpec((1, H), lambda i: (0, 0)),
        ],
        out_specs=pl.BlockSpec((_RB, H), lambda i: (i, 0)),
        out_shape=jax.ShapeDtypeStruct((N, H), _f32),
    )(x, parts, w1, b12, w2, b22)


def _pool_body(x_ref, b3_ref, wp1_ref, bp1_ref, wp2_ref, bp2_ref, o_ref,
               acc_ref):
    i = pl.program_id(0)

    @pl.when(i == 0)
    def _():
        acc_ref[...] = jnp.zeros_like(acc_ref)

    brow = b3_ref[0]
    rows = lax.broadcasted_iota(jnp.int32, (G, _RB), 0)
    oh = (rows == brow).astype(_f32)
    acc_ref[...] += jnp.dot(oh, x_ref[...], preferred_element_type=_f32,
                            precision=lax.Precision.HIGHEST)

    @pl.when(i == _NB - 1)
    def _():
        g = acc_ref[...]
        h = jnp.maximum(
            jnp.dot(g, wp1_ref[...], preferred_element_type=_f32)
            + bp1_ref[...], 0.0)
        o_ref[...] = jnp.dot(h, wp2_ref[...], preferred_element_type=_f32
                             ) + bp2_ref[...]


def _pool_call(x, batch3, wp1, bp12, wp2pad, bp2pad):
    return pl.pallas_call(
        _pool_body,
        grid=(_NB,),
        in_specs=[
            pl.BlockSpec((_RB, H), lambda i: (i, 0)),
            pl.BlockSpec((1, 1, _RB), lambda i: (i, 0, 0)),
            pl.BlockSpec((F, F), lambda i: (0, 0)),
            pl.BlockSpec((1, F), lambda i: (0, 0)),
            pl.BlockSpec((F, F), lambda i: (0, 0)),
            pl.BlockSpec((1, F), lambda i: (0, 0)),
        ],
        out_specs=pl.BlockSpec((G, F), lambda i: (0, 0)),
        out_shape=jax.ShapeDtypeStruct((G, F), _f32),
        scratch_shapes=[pltpu.VMEM((G, F), _f32)],
    )(x, batch3, wp1, bp12, wp2pad, bp2pad)


def kernel(z, pos, edge_index, batch, embed, W_pos, b_pos, W_comb, b_comb,
           gin0_W1, gin0_b1, gin0_W2, gin0_b2,
           gin1_W1, gin1_b1, gin1_W2, gin1_b2,
           gin2_W1, gin2_b1, gin2_W2, gin2_b2,
           Wp1, bp1, Wp2, bp2):
    src = edge_index[0].astype(jnp.int32)
    dst = edge_index[1].astype(jnp.int32)
    pad = EPAD - E
    packed = jnp.left_shift(dst, SHIFT) | src
    t = jnp.arange(pad, dtype=jnp.int32)
    pad_packed = jnp.left_shift(N + (t % NSINK), SHIFT) | (t % N)
    epk = jnp.concatenate([packed, pad_packed]).reshape(NCH, CHUNK)
    z2 = z.astype(jnp.int32).reshape(N, 1)
    pos8 = jnp.pad(pos, ((0, 0), (0, 8 - pos.shape[1])))
    emb_pad = jnp.pad(embed, ((0, 128 - embed.shape[0]), (0, 0)))
    wpos8 = jnp.pad(W_pos, ((0, 8 - W_pos.shape[0]), (0, 0)))
    zeros = jnp.zeros((NPAD, H), _f32)
    batch3 = batch.astype(jnp.int32).reshape(_NB, 1, _RB)
    wp2pad = jnp.pad(Wp2, ((0, 0), (0, F - Wp2.shape[1])))
    bp2pad = jnp.pad(bp2, ((0, F - bp2.shape[0]),)).reshape(1, F)

    x = _x0_call(z2, pos8, emb_pad, wpos8, b_pos.reshape(1, H),
                 W_comb[:H], W_comb[H:], b_comb.reshape(1, H))

    gin = [(gin0_W1, gin0_b1, gin0_W2, gin0_b2),
           (gin1_W1, gin1_b1, gin1_W2, gin1_b2),
           (gin2_W1, gin2_b1, gin2_W2, gin2_b2)]
    for l, (w1, b1, w2, b2) in enumerate(gin):
        parts = _edge_aggr(x, epk, zeros)
        x = _mlp_call(x, parts, w1, b1.reshape(1, H), w2, b2.reshape(1, H),
                      final_relu=(l < 2))

    out = _pool_call(x, batch3, Wp1, bp1.reshape(1, F), wp2pad, bp2pad)
    return out[:, :1]
